# baseline (speedup 1.0000x reference)
import functools

import jax
import jax.numpy as jnp
from jax import lax
from jax.experimental import pallas as pl
from jax.experimental.pallas import tpu as pltpu
from jax.experimental.pallas import tpu_sc as plsc

_K = 16


def _sc_gather(table, idx, chunk=512):
    R = idx.shape[0]
    D = table.shape[1]
    info = plsc.get_sparse_core_info()
    nw = info.num_cores * info.num_subcores
    per_w = R // nw
    ch = min(chunk, per_w)
    mesh = plsc.VectorSubcoreMesh(core_axis_name="c", subcore_axis_name="s")

    @functools.partial(
        pl.kernel, mesh=mesh,
        out_type=jax.ShapeDtypeStruct((R, D), jnp.float32),
        scratch_types=[
            pltpu.VMEM((ch,), jnp.int32),
            pltpu.VMEM((ch, D), jnp.float32),
            pltpu.SemaphoreType.DMA,
        ],
    )
    def gk(table_hbm, idx_hbm, out_hbm, idx_v, rows_v, sem):
        wid = lax.axis_index("s") * info.num_cores + lax.axis_index("c")
        base = wid * per_w

        def body(i, carry):
            off = base + i * ch
            pltpu.sync_copy(idx_hbm.at[pl.ds(off, ch)], idx_v)
            pltpu.async_copy(table_hbm.at[idx_v], rows_v, sem).wait()
            pltpu.sync_copy(rows_v, out_hbm.at[pl.ds(off, ch)])
            return carry

        lax.fori_loop(0, per_w // ch, body, 0)

    return gk(table, idx)


def _knn_body(K, M, q_ref, st_ref, pen_ref, o_ref):
    q = q_ref[0]
    qq = jnp.sum(q * q, axis=1, keepdims=True)
    st = st_ref[0]
    s0, s1, s2 = st[0:1], st[1:2], st[2:3]
    ss = s0 * s0 + s1 * s1 + s2 * s2
    pen = pen_ref[0]
    cross = jax.lax.dot_general(
        q, st, dimension_numbers=(((1,), (0,)), ((), ())),
        preferred_element_type=jnp.float32)
    d = qq + ss - 2.0 * cross + pen
    iota = jax.lax.broadcasted_iota(jnp.int32, (1, M), 1)
    cols = []
    for _ in range(K):
        v = jnp.min(d, axis=1, keepdims=True)
        i = jnp.min(jnp.where(d == v, iota, M), axis=1, keepdims=True)
        cols.append(i)
        d = jnp.where(iota == i, jnp.inf, d)
    o_ref[0] = jnp.concatenate(cols, axis=1)


def _knn_topk(q, s, pen, K, TQ=128):
    B, NQ, _ = q.shape
    M = s.shape[1]
    st = jnp.transpose(s, (0, 2, 1))
    pen3 = pen.reshape(B, 1, M)
    body = functools.partial(_knn_body, K, M)
    return pl.pallas_call(
        body,
        grid=(B, NQ // TQ),
        in_specs=[
            pl.BlockSpec((1, TQ, 3), lambda b, j: (b, j, 0)),
            pl.BlockSpec((1, 3, M), lambda b, j: (b, 0, 0)),
            pl.BlockSpec((1, 1, M), lambda b, j: (b, 0, 0)),
        ],
        out_specs=pl.BlockSpec((1, TQ, K), lambda b, j: (b, j, 0)),
        out_shape=jax.ShapeDtypeStruct((B, NQ, K), jnp.int32),
    )(q, st, pen3)


_TN = 512


def _first():
    return jnp.logical_and(pl.program_id(0) == 0, pl.program_id(1) == 0)


def _lrelu(z):
    return jnp.where(z >= 0, z, 0.01 * z)


def _mask2(mask_ref, TN):
    m = mask_ref[0]
    return jnp.broadcast_to(m[:, None, :], (TN, 16, 1)).reshape(TN * 16, 1)


def _gather_parts(descs, refs, TN):
    R = TN * 16
    parts = []
    i = 0
    for d in descs:
        if d[0] == 'y':
            a = refs[i][0].reshape(R, 64)
            s = refs[i + 1][0, 0, :]
            bb = refs[i + 2][0, 0, :]
            parts.append(_lrelu(a * s + bb))
            i += 3
        elif d[0] == 'raw4':
            lo, hi = d[2], d[3]
            parts.append(refs[i][0][:, :, lo:hi].reshape(R, hi - lo))
            i += 1
        else:
            a = refs[i][0]
            C = a.shape[-1]
            parts.append(jnp.broadcast_to(a[:, None, :], (TN, 16, C)).reshape(R, C))
            i += 1
    return parts


def _in_specs_for(descs, TN):
    specs = []
    for d in descs:
        if d[0] == 'y':
            specs += [pl.BlockSpec((1, TN, 16, 64), lambda b, j: (b, j, 0, 0)),
                      pl.BlockSpec((1, 1, 64), lambda b, j: (0, 0, 0)),
                      pl.BlockSpec((1, 1, 64), lambda b, j: (0, 0, 0))]
        elif d[0] == 'raw4':
            specs.append(pl.BlockSpec((1, TN, 16, 128), lambda b, j: (b, j, 0, 0)))
        else:
            C = d[1].shape[-1]
            specs.append(pl.BlockSpec((1, TN, C), lambda b, j: (b, j, 0)))
    return specs


def _in_arrays_for(descs):
    arrs = []
    for d in descs:
        if d[0] == 'y':
            arrs += [d[1], d[2].reshape(1, 1, 64), d[3].reshape(1, 1, 64)]
        else:
            arrs.append(d[1])
    return arrs


def _mm_pass(descs, Wt, mask3, TN=_TN):
    first4 = next(d[1] for d in descs if d[0] != 'b3')
    B, N = first4.shape[0], first4.shape[1]
    cin = Wt.shape[0]

    def body(*refs):
        mask_ref, w_ref = refs[0], refs[1]
        y_ref, acc_ref = refs[-2], refs[-1]
        m2 = _mask2(mask_ref, TN)
        parts = _gather_parts(descs, refs[2:-2], TN)
        X = parts[0] if len(parts) == 1 else jnp.concatenate(parts, axis=1)
        Y = lax.dot_general(X, w_ref[...], (((1,), (0,)), ((), ())),
                            preferred_element_type=jnp.float32)
        y_ref[0] = Y.reshape(TN, 16, 64)

        @pl.when(_first())
        def _():
            acc_ref[...] = jnp.zeros((1, 1, 64), jnp.float32)
        acc_ref[...] += jnp.sum(Y * m2, axis=0).reshape(1, 1, 64)

    return pl.pallas_call(
        body,
        grid=(B, N // TN),
        in_specs=[pl.BlockSpec((1, TN, 1), lambda b, j: (0, j, 0)),
                  pl.BlockSpec((cin, 64), lambda b, j: (0, 0))]
        + _in_specs_for(descs, TN),
        out_specs=[pl.BlockSpec((1, TN, 16, 64), lambda b, j: (b, j, 0, 0)),
                   pl.BlockSpec((1, 1, 64), lambda b, j: (0, 0, 0))],
        out_shape=[jax.ShapeDtypeStruct((B, N, 16, 64), jnp.float32),
                   jax.ShapeDtypeStruct((1, 1, 64), jnp.float32)],
    )(mask3, Wt, *_in_arrays_for(descs))


def _var_pass(y, mean, mask3, TN=_TN):
    B, N = y.shape[0], y.shape[1]

    def body(mask_ref, mean_ref, y_ref, acc_ref):
        m2 = _mask2(mask_ref, TN)
        z = y_ref[0].reshape(TN * 16, 64) - mean_ref[0, 0, :]

        @pl.when(_first())
        def _():
            acc_ref[...] = jnp.zeros((1, 1, 64), jnp.float32)
        acc_ref[...] += jnp.sum(z * z * m2, axis=0).reshape(1, 1, 64)

    return pl.pallas_call(
        body,
        grid=(B, N // TN),
        in_specs=[pl.BlockSpec((1, TN, 1), lambda b, j: (0, j, 0)),
                  pl.BlockSpec((1, 1, 64), lambda b, j: (0, 0, 0)),
                  pl.BlockSpec((1, TN, 16, 64), lambda b, j: (b, j, 0, 0))],
        out_specs=pl.BlockSpec((1, 1, 64), lambda b, j: (0, 0, 0)),
        out_shape=jax.ShapeDtypeStruct((1, 1, 64), jnp.float32),
    )(mask3, mean.reshape(1, 1, 64), y)


def _bn_affine(sums, sumsq_fn, y, mask3, denom, gamma, beta):
    mean = sums.reshape(64) / denom
    var = sumsq_fn(y, mean, mask3).reshape(64) / denom
    s = gamma / jnp.sqrt(var + 1e-5)
    return s, beta - mean * s


def _softmax_wsum(logits_parts, feats_parts):
    m = logits_parts[0]
    for zk in logits_parts[1:]:
        m = jnp.maximum(m, zk)
    es = [jnp.exp(zk - m) for zk in logits_parts]
    s = es[0]
    for ek in es[1:]:
        s = s + ek
    tot = (es[0] / s) * feats_parts[0]
    for ek, fk in zip(es[1:], feats_parts[1:]):
        tot = tot + (ek / s) * fk
    return tot


def _out_pass(wq_y, wq_s, wq_b, feat_mode, feat_arr, f_s, f_b, TN=_TN):
    B, N = wq_y.shape[0], wq_y.shape[1]
    fspec = (pl.BlockSpec((1, TN, 16, 64), lambda b, j: (b, j, 0, 0))
             if feat_mode == 'y' else
             pl.BlockSpec((1, TN, 16, 128), lambda b, j: (b, j, 0, 0)))

    def body(y_ref, s_ref, b_ref, f_ref, fs_ref, fb_ref, o_ref):
        z = _lrelu(y_ref[0] * s_ref[0, 0, :] + b_ref[0, 0, :])
        if feat_mode == 'y':
            f = _lrelu(f_ref[0] * fs_ref[0, 0, :] + fb_ref[0, 0, :])
        else:
            f = f_ref[0][:, :, 0:64]
        zs = [z[:, k:k + 1, :] for k in range(16)]
        fs = [f[:, k:k + 1, :] for k in range(16)]
        o_ref[0] = _softmax_wsum(zs, fs).reshape(TN, 64)

    return pl.pallas_call(
        body,
        grid=(B, N // TN),
        in_specs=[pl.BlockSpec((1, TN, 16, 64), lambda b, j: (b, j, 0, 0)),
                  pl.BlockSpec((1, 1, 64), lambda b, j: (0, 0, 0)),
                  pl.BlockSpec((1, 1, 64), lambda b, j: (0, 0, 0)),
                  fspec,
                  pl.BlockSpec((1, 1, 64), lambda b, j: (0, 0, 0)),
                  pl.BlockSpec((1, 1, 64), lambda b, j: (0, 0, 0))],
        out_specs=pl.BlockSpec((1, TN, 64), lambda b, j: (b, j, 0)),
        out_shape=jax.ShapeDtypeStruct((B, N, 64), jnp.float32),
    )(wq_y, wq_s.reshape(1, 1, 64), wq_b.reshape(1, 1, 64),
      feat_arr, f_s.reshape(1, 1, 64), f_b.reshape(1, 1, 64))


def _build_pi_pass(g1, wxlz, wpn, W1t, Wet, mask3, TN=_TN):
    B, N = g1.shape[0], g1.shape[1]
    R = TN * 16

    def body(mask_ref, w1_ref, we_ref, g_ref, wx_ref, wpn_ref,
             y1_ref, ye_ref, a1_ref, ae_ref):
        m2 = _mask2(mask_ref, TN)
        g = g_ref[0]
        gf = g[:, :, 0:64]
        gx = g[:, :, 64:67].reshape(R, 3)
        mu = jnp.mean(gf, axis=-1, keepdims=True)
        d = gf - mu
        sd = jnp.sqrt(jnp.sum(d * d, axis=-1, keepdims=True) / 63.0)
        gfn = d / jnp.maximum(sd, 1e-12)
        wx3 = jnp.broadcast_to(wx_ref[0][:, None, :], (TN, 16, 3)).reshape(R, 3)
        wpb = jnp.broadcast_to(wpn_ref[0][:, None, :], (TN, 16, 64))
        X70 = jnp.concatenate([wx3, gx, (wpb * gfn).reshape(R, 64)], axis=1)
        X6 = jnp.concatenate([wx3, gx], axis=1)
        Y1 = lax.dot_general(X70, w1_ref[...], (((1,), (0,)), ((), ())),
                             preferred_element_type=jnp.float32)
        Ye = lax.dot_general(X6, we_ref[...], (((1,), (0,)), ((), ())),
                             preferred_element_type=jnp.float32)
        y1_ref[0] = Y1.reshape(TN, 16, 64)
        ye_ref[0] = Ye.reshape(TN, 16, 64)

        @pl.when(_first())
        def _():
            a1_ref[...] = jnp.zeros((1, 1, 64), jnp.float32)
            ae_ref[...] = jnp.zeros((1, 1, 64), jnp.float32)
        a1_ref[...] += jnp.sum(Y1 * m2, axis=0).reshape(1, 1, 64)
        ae_ref[...] += jnp.sum(Ye * m2, axis=0).reshape(1, 1, 64)

    return pl.pallas_call(
        body,
        grid=(B, N // TN),
        in_specs=[pl.BlockSpec((1, TN, 1), lambda b, j: (0, j, 0)),
                  pl.BlockSpec((70, 64), lambda b, j: (0, 0)),
                  pl.BlockSpec((6, 64), lambda b, j: (0, 0)),
                  pl.BlockSpec((1, TN, 16, 128), lambda b, j: (b, j, 0, 0)),
                  pl.BlockSpec((1, TN, 3), lambda b, j: (b, j, 0)),
                  pl.BlockSpec((1, TN, 64), lambda b, j: (b, j, 0))],
        out_specs=[pl.BlockSpec((1, TN, 16, 64), lambda b, j: (b, j, 0, 0)),
                   pl.BlockSpec((1, TN, 16, 64), lambda b, j: (b, j, 0, 0)),
                   pl.BlockSpec((1, 1, 64), lambda b, j: (0, 0, 0)),
                   pl.BlockSpec((1, 1, 64), lambda b, j: (0, 0, 0))],
        out_shape=[jax.ShapeDtypeStruct((B, N, 16, 64), jnp.float32),
                   jax.ShapeDtypeStruct((B, N, 16, 64), jnp.float32),
                   jax.ShapeDtypeStruct((1, 1, 64), jnp.float32),
                   jax.ShapeDtypeStruct((1, 1, 64), jnp.float32)],
    )(mask3, W1t, Wet, g1, wxlz, wpn)


def _build_pc_pass(g2, wxlz, Wpct, mask3, TN=_TN):
    B, N = g2.shape[0], g2.shape[1]
    R = TN * 16

    def body(mask_ref, w_ref, g_ref, wx_ref, ye_ref, acc_ref):
        m2 = _mask2(mask_ref, TN)
        gxp = g_ref[0][:, :, 64:67]
        wx3 = jnp.broadcast_to(wx_ref[0][:, None, :], (TN, 16, 3))
        diff = gxp - wx3
        euc = jnp.sqrt(jnp.sum(diff * diff, axis=2, keepdims=True) + 1e-20)
        X10 = jnp.concatenate([wx3, gxp, diff, euc], axis=2).reshape(R, 10)
        Y = lax.dot_general(X10, w_ref[...], (((1,), (0,)), ((), ())),
                            preferred_element_type=jnp.float32)
        ye_ref[0] = Y.reshape(TN, 16, 64)

        @pl.when(_first())
        def _():
            acc_ref[...] = jnp.zeros((1, 1, 64), jnp.float32)
        acc_ref[...] += jnp.sum(Y * m2, axis=0).reshape(1, 1, 64)

    return pl.pallas_call(
        body,
        grid=(B, N // TN),
        in_specs=[pl.BlockSpec((1, TN, 1), lambda b, j: (0, j, 0)),
                  pl.BlockSpec((10, 64), lambda b, j: (0, 0)),
                  pl.BlockSpec((1, TN, 16, 128), lambda b, j: (b, j, 0, 0)),
                  pl.BlockSpec((1, TN, 3), lambda b, j: (b, j, 0))],
        out_specs=[pl.BlockSpec((1, TN, 16, 64), lambda b, j: (b, j, 0, 0)),
                   pl.BlockSpec((1, 1, 64), lambda b, j: (0, 0, 0))],
        out_shape=[jax.ShapeDtypeStruct((B, N, 16, 64), jnp.float32),
                   jax.ShapeDtypeStruct((1, 1, 64), jnp.float32)],
    )(mask3, Wpct, g2, wxlz)


def _nc2bnc(feats, batch_info, length):
    counts = jnp.bincount(batch_info, length=length)
    n = feats[0].shape[0]
    n_t = jnp.max(counts)
    offset = jnp.cumsum(counts) - counts
    ind = jnp.arange(n)
    new_count = jnp.full_like(counts, n)
    new_offset = jnp.cumsum(new_count) - new_count
    ind = ind + (new_offset - offset)[batch_info]
    out = []
    for feat in feats:
        c = feat.shape[-1]
        buf = jnp.zeros((length * n, c), dtype=jnp.float32).at[ind].set(feat)
        out.append(buf.reshape(length, n, c))
    return out, ind, n_t


def _norm_ch(x):
    m = jnp.mean(x, axis=-1, keepdims=True)
    s = jnp.std(x, axis=-1, keepdims=True, ddof=1)
    return (x - m) / jnp.maximum(s, 1e-12)


def kernel(warped_xyz, warped_points, batch_info, batch_size, f2_xyz, f2_points, lidar_z, params):
    B = f2_xyz.shape[0]
    M = f2_xyz.shape[1]
    (wx, wp, lz), inv, n_t = _nc2bnc(
        [warped_xyz, warped_points, lidar_z], batch_info, B)
    valid_mask = (jnp.sum(wx * wx, axis=-1) >= 1e-10).astype(jnp.float32)

    idx_q = _knn_topk(wx, f2_xyz, jnp.zeros((B, M), jnp.float32), _K)
    bi = jnp.arange(B)[:, None, None]
    C = f2_points.shape[-1]
    n_all = wx.shape[1]
    tab1 = jnp.concatenate(
        [f2_points.reshape(B * M, C),
         jnp.pad(f2_xyz.reshape(B * M, 3), ((0, 0), (0, 125 - C)))], axis=1)
    flat_q = (idx_q + (bi * M)).reshape(-1)
    g1 = _sc_gather(tab1, flat_q).reshape(B, n_all, _K, 128)

    wxlz = wx * lz
    n_f = n_t.astype(jnp.float32)
    denom = B * n_f * _K
    wpn = _norm_ch(wp)
    mask3 = (jnp.arange(n_all) < n_t).astype(jnp.float32).reshape(1, n_all, 1)

    p1, p2 = params['mlp1']
    pe = params['pi_enc']
    p3, p4 = params['mlp2']
    ppc = params['pc_enc']
    p5, p6 = params['mlp2b']

    y1, ye, sum1, sume = _build_pi_pass(g1, wxlz, wpn, p1[0].T, pe[0].T, mask3)
    s1, b1 = _bn_affine(sum1, _var_pass, y1, mask3, denom, p1[1], p1[2])
    se, be = _bn_affine(sume, _var_pass, ye, mask3, denom, pe[1], pe[2])
    y2, sum2 = _mm_pass([('y', y1, s1, b1)], p2[0].T, mask3)
    s2, b2 = _bn_affine(sum2, _var_pass, y2, mask3, denom, p2[1], p2[2])
    y3, sum3 = _mm_pass([('y', ye, se, be), ('y', y2, s2, b2)], p3[0].T, mask3)
    s3, b3 = _bn_affine(sum3, _var_pass, y3, mask3, denom, p3[1], p3[2])
    y4, sum4 = _mm_pass([('y', y3, s3, b3)], p4[0].T, mask3)
    s4, b4 = _bn_affine(sum4, _var_pass, y4, mask3, denom, p4[1], p4[2])
    pi_feat1_new = _out_pass(y4, s4, b4, 'y', y2, s2, b2)

    pen = (1.0 - valid_mask) * 1e10
    idx_p = _knn_topk(wxlz, wxlz, pen, _K)
    tab2 = jnp.concatenate(
        [pi_feat1_new.reshape(B * n_all, 64),
         jnp.pad(wxlz.reshape(B * n_all, 3), ((0, 0), (0, 61)))], axis=1)
    flat_p = (idx_p + (bi * n_all)).reshape(-1)
    g2 = _sc_gather(tab2, flat_p).reshape(B, n_all, _K, 128)

    yenc, sumenc = _build_pc_pass(g2, wxlz, ppc[0].T, mask3)
    senc, benc = _bn_affine(sumenc, _var_pass, yenc, mask3, denom, ppc[1], ppc[2])
    y5, sum5 = _mm_pass(
        [('y', yenc, senc, benc), ('b3', wp), ('raw4', g2, 0, 64)], p5[0].T, mask3)
    s5, b5 = _bn_affine(sum5, _var_pass, y5, mask3, denom, p5[1], p5[2])
    y6, sum6 = _mm_pass([('y', y5, s5, b5)], p6[0].T, mask3)
    s6, b6 = _bn_affine(sum6, _var_pass, y6, mask3, denom, p6[1], p6[2])
    pc_feat1_new = _out_pass(y6, s6, b6, 'raw4', g2, s6, b6)

    flat_out = jnp.pad(pc_feat1_new.reshape(B * n_all, 64), ((0, 0), (0, 64)))
    return _sc_gather(flat_out, inv.astype(jnp.int32), chunk=256)[:, :64]


# speedup vs baseline: 1.1084x; 1.1084x over previous
import functools

import jax
import jax.numpy as jnp
from jax import lax
from jax.experimental import pallas as pl
from jax.experimental.pallas import tpu as pltpu
from jax.experimental.pallas import tpu_sc as plsc

_K = 16


def _sc_gather(table, idx, chunk=512):
    R = idx.shape[0]
    D = table.shape[1]
    info = plsc.get_sparse_core_info()
    nw = info.num_cores * info.num_subcores
    per_w = R // nw
    ch = min(chunk, per_w)
    mesh = plsc.VectorSubcoreMesh(core_axis_name="c", subcore_axis_name="s")

    @functools.partial(
        pl.kernel, mesh=mesh,
        out_type=jax.ShapeDtypeStruct((R, D), jnp.float32),
        scratch_types=[
            pltpu.VMEM((ch,), jnp.int32),
            pltpu.VMEM((ch, D), jnp.float32),
            pltpu.SemaphoreType.DMA,
        ],
    )
    def gk(table_hbm, idx_hbm, out_hbm, idx_v, rows_v, sem):
        wid = lax.axis_index("s") * info.num_cores + lax.axis_index("c")
        base = wid * per_w

        def body(i, carry):
            off = base + i * ch
            pltpu.sync_copy(idx_hbm.at[pl.ds(off, ch)], idx_v)
            pltpu.async_copy(table_hbm.at[idx_v], rows_v, sem).wait()
            pltpu.sync_copy(rows_v, out_hbm.at[pl.ds(off, ch)])
            return carry

        lax.fori_loop(0, per_w // ch, body, 0)

    return gk(table, idx)


def _knn_body(K, M, TQ, nt_ref, q_ref, st_ref, pen_ref, o_ref):
    blk = pl.program_id(1) * TQ

    @pl.when(blk < nt_ref[0])
    def _():
        q = q_ref[0]
        qq = jnp.sum(q * q, axis=1, keepdims=True)
        st = st_ref[0]
        s0, s1, s2 = st[0:1], st[1:2], st[2:3]
        ss = s0 * s0 + s1 * s1 + s2 * s2
        pen = pen_ref[0]
        cross = jax.lax.dot_general(
            q, st, dimension_numbers=(((1,), (0,)), ((), ())),
            preferred_element_type=jnp.float32)
        d = qq + ss - 2.0 * cross + pen
        iota = jax.lax.broadcasted_iota(jnp.int32, (1, M), 1)
        cols = []
        for _ in range(K):
            v = jnp.min(d, axis=1, keepdims=True)
            i = jnp.min(jnp.where(d == v, iota, M), axis=1, keepdims=True)
            cols.append(i)
            d = jnp.where(iota == i, jnp.inf, d)
        o_ref[0] = jnp.concatenate(cols, axis=1)

    @pl.when(blk >= nt_ref[0])
    def _():
        o_ref[0] = jnp.zeros((TQ, K), jnp.int32)


def _knn_topk(q, s, pen, n_t, K, TQ=128):
    B, NQ, _ = q.shape
    M = s.shape[1]
    st = jnp.transpose(s, (0, 2, 1))
    pen3 = pen.reshape(B, 1, M)
    body = functools.partial(_knn_body, K, M, TQ)
    grid_spec = pltpu.PrefetchScalarGridSpec(
        num_scalar_prefetch=1,
        grid=(B, NQ // TQ),
        in_specs=[
            pl.BlockSpec((1, TQ, 3), lambda b, j, nt: (b, j, 0)),
            pl.BlockSpec((1, 3, M), lambda b, j, nt: (b, 0, 0)),
            pl.BlockSpec((1, 1, M), lambda b, j, nt: (b, 0, 0)),
        ],
        out_specs=pl.BlockSpec((1, TQ, K), lambda b, j, nt: (b, j, 0)),
    )
    return pl.pallas_call(
        body,
        grid_spec=grid_spec,
        out_shape=jax.ShapeDtypeStruct((B, NQ, K), jnp.int32),
    )(jnp.reshape(n_t, (1,)).astype(jnp.int32), q, st, pen3)


_TN = 512


def _first():
    return jnp.logical_and(pl.program_id(0) == 0, pl.program_id(1) == 0)


def _lrelu(z):
    return jnp.where(z >= 0, z, 0.01 * z)


def _mask2(mask_ref, TN):
    m = mask_ref[0]
    return jnp.broadcast_to(m[:, None, :], (TN, 16, 1)).reshape(TN * 16, 1)


def _gather_parts(descs, refs, TN):
    R = TN * 16
    parts = []
    i = 0
    for d in descs:
        if d[0] == 'y':
            a = refs[i][0].reshape(R, 64)
            s = refs[i + 1][0, 0, :]
            bb = refs[i + 2][0, 0, :]
            parts.append(_lrelu(a * s + bb))
            i += 3
        elif d[0] == 'raw4':
            lo, hi = d[2], d[3]
            parts.append(refs[i][0][:, :, lo:hi].reshape(R, hi - lo))
            i += 1
        else:
            a = refs[i][0]
            C = a.shape[-1]
            parts.append(jnp.broadcast_to(a[:, None, :], (TN, 16, C)).reshape(R, C))
            i += 1
    return parts


def _in_specs_for(descs, TN):
    specs = []
    for d in descs:
        if d[0] == 'y':
            specs += [pl.BlockSpec((1, TN, 16, 64), lambda b, j: (b, j, 0, 0)),
                      pl.BlockSpec((1, 1, 64), lambda b, j: (0, 0, 0)),
                      pl.BlockSpec((1, 1, 64), lambda b, j: (0, 0, 0))]
        elif d[0] == 'raw4':
            specs.append(pl.BlockSpec((1, TN, 16, 128), lambda b, j: (b, j, 0, 0)))
        else:
            C = d[1].shape[-1]
            specs.append(pl.BlockSpec((1, TN, C), lambda b, j: (b, j, 0)))
    return specs


def _in_arrays_for(descs):
    arrs = []
    for d in descs:
        if d[0] == 'y':
            arrs += [d[1], d[2].reshape(1, 1, 64), d[3].reshape(1, 1, 64)]
        else:
            arrs.append(d[1])
    return arrs


def _mm_pass(descs, Wt, mask3, TN=_TN):
    first4 = next(d[1] for d in descs if d[0] != 'b3')
    B, N = first4.shape[0], first4.shape[1]
    cin = Wt.shape[0]

    def body(*refs):
        mask_ref, w_ref = refs[0], refs[1]
        y_ref, acc_ref = refs[-2], refs[-1]
        m2 = _mask2(mask_ref, TN)
        parts = _gather_parts(descs, refs[2:-2], TN)
        X = parts[0] if len(parts) == 1 else jnp.concatenate(parts, axis=1)
        Y = lax.dot_general(X, w_ref[...], (((1,), (0,)), ((), ())),
                            preferred_element_type=jnp.float32)
        y_ref[0] = Y.reshape(TN, 16, 64)

        @pl.when(_first())
        def _():
            acc_ref[...] = jnp.zeros((1, 1, 64), jnp.float32)
        acc_ref[...] += jnp.sum(Y * m2, axis=0).reshape(1, 1, 64)

    return pl.pallas_call(
        body,
        grid=(B, N // TN),
        in_specs=[pl.BlockSpec((1, TN, 1), lambda b, j: (0, j, 0)),
                  pl.BlockSpec((cin, 64), lambda b, j: (0, 0))]
        + _in_specs_for(descs, TN),
        out_specs=[pl.BlockSpec((1, TN, 16, 64), lambda b, j: (b, j, 0, 0)),
                   pl.BlockSpec((1, 1, 64), lambda b, j: (0, 0, 0))],
        out_shape=[jax.ShapeDtypeStruct((B, N, 16, 64), jnp.float32),
                   jax.ShapeDtypeStruct((1, 1, 64), jnp.float32)],
    )(mask3, Wt, *_in_arrays_for(descs))


def _var_pass(y, mean, mask3, TN=_TN):
    B, N = y.shape[0], y.shape[1]

    def body(mask_ref, mean_ref, y_ref, acc_ref):
        m2 = _mask2(mask_ref, TN)
        z = y_ref[0].reshape(TN * 16, 64) - mean_ref[0, 0, :]

        @pl.when(_first())
        def _():
            acc_ref[...] = jnp.zeros((1, 1, 64), jnp.float32)
        acc_ref[...] += jnp.sum(z * z * m2, axis=0).reshape(1, 1, 64)

    return pl.pallas_call(
        body,
        grid=(B, N // TN),
        in_specs=[pl.BlockSpec((1, TN, 1), lambda b, j: (0, j, 0)),
                  pl.BlockSpec((1, 1, 64), lambda b, j: (0, 0, 0)),
                  pl.BlockSpec((1, TN, 16, 64), lambda b, j: (b, j, 0, 0))],
        out_specs=pl.BlockSpec((1, 1, 64), lambda b, j: (0, 0, 0)),
        out_shape=jax.ShapeDtypeStruct((1, 1, 64), jnp.float32),
    )(mask3, mean.reshape(1, 1, 64), y)


def _bn_affine(sums, sumsq_fn, y, mask3, denom, gamma, beta):
    mean = sums.reshape(64) / denom
    var = sumsq_fn(y, mean, mask3).reshape(64) / denom
    s = gamma / jnp.sqrt(var + 1e-5)
    return s, beta - mean * s


def _softmax_wsum(logits_parts, feats_parts):
    m = logits_parts[0]
    for zk in logits_parts[1:]:
        m = jnp.maximum(m, zk)
    es = [jnp.exp(zk - m) for zk in logits_parts]
    s = es[0]
    for ek in es[1:]:
        s = s + ek
    tot = (es[0] / s) * feats_parts[0]
    for ek, fk in zip(es[1:], feats_parts[1:]):
        tot = tot + (ek / s) * fk
    return tot


def _out_pass(wq_y, wq_s, wq_b, feat_mode, feat_arr, f_s, f_b, TN=_TN):
    B, N = wq_y.shape[0], wq_y.shape[1]
    fspec = (pl.BlockSpec((1, TN, 16, 64), lambda b, j: (b, j, 0, 0))
             if feat_mode == 'y' else
             pl.BlockSpec((1, TN, 16, 128), lambda b, j: (b, j, 0, 0)))

    def body(y_ref, s_ref, b_ref, f_ref, fs_ref, fb_ref, o_ref):
        z = _lrelu(y_ref[0] * s_ref[0, 0, :] + b_ref[0, 0, :])
        if feat_mode == 'y':
            f = _lrelu(f_ref[0] * fs_ref[0, 0, :] + fb_ref[0, 0, :])
        else:
            f = f_ref[0][:, :, 0:64]
        zs = [z[:, k:k + 1, :] for k in range(16)]
        fs = [f[:, k:k + 1, :] for k in range(16)]
        o_ref[0] = _softmax_wsum(zs, fs).reshape(TN, 64)

    return pl.pallas_call(
        body,
        grid=(B, N // TN),
        in_specs=[pl.BlockSpec((1, TN, 16, 64), lambda b, j: (b, j, 0, 0)),
                  pl.BlockSpec((1, 1, 64), lambda b, j: (0, 0, 0)),
                  pl.BlockSpec((1, 1, 64), lambda b, j: (0, 0, 0)),
                  fspec,
                  pl.BlockSpec((1, 1, 64), lambda b, j: (0, 0, 0)),
                  pl.BlockSpec((1, 1, 64), lambda b, j: (0, 0, 0))],
        out_specs=pl.BlockSpec((1, TN, 64), lambda b, j: (b, j, 0)),
        out_shape=jax.ShapeDtypeStruct((B, N, 64), jnp.float32),
    )(wq_y, wq_s.reshape(1, 1, 64), wq_b.reshape(1, 1, 64),
      feat_arr, f_s.reshape(1, 1, 64), f_b.reshape(1, 1, 64))


def _build_pi_pass(g1, wxlz, wpn, W1t, Wet, mask3, TN=_TN):
    B, N = g1.shape[0], g1.shape[1]
    R = TN * 16

    def body(mask_ref, w1_ref, we_ref, g_ref, wx_ref, wpn_ref,
             y1_ref, ye_ref, a1_ref, ae_ref):
        m2 = _mask2(mask_ref, TN)
        g = g_ref[0]
        gf = g[:, :, 0:64]
        gx = g[:, :, 64:67].reshape(R, 3)
        mu = jnp.mean(gf, axis=-1, keepdims=True)
        d = gf - mu
        sd = jnp.sqrt(jnp.sum(d * d, axis=-1, keepdims=True) / 63.0)
        gfn = d / jnp.maximum(sd, 1e-12)
        wx3 = jnp.broadcast_to(wx_ref[0][:, None, :], (TN, 16, 3)).reshape(R, 3)
        wpb = jnp.broadcast_to(wpn_ref[0][:, None, :], (TN, 16, 64))
        X70 = jnp.concatenate([wx3, gx, (wpb * gfn).reshape(R, 64)], axis=1)
        X6 = jnp.concatenate([wx3, gx], axis=1)
        Y1 = lax.dot_general(X70, w1_ref[...], (((1,), (0,)), ((), ())),
                             preferred_element_type=jnp.float32)
        Ye = lax.dot_general(X6, we_ref[...], (((1,), (0,)), ((), ())),
                             preferred_element_type=jnp.float32)
        y1_ref[0] = Y1.reshape(TN, 16, 64)
        ye_ref[0] = Ye.reshape(TN, 16, 64)

        @pl.when(_first())
        def _():
            a1_ref[...] = jnp.zeros((1, 1, 64), jnp.float32)
            ae_ref[...] = jnp.zeros((1, 1, 64), jnp.float32)
        a1_ref[...] += jnp.sum(Y1 * m2, axis=0).reshape(1, 1, 64)
        ae_ref[...] += jnp.sum(Ye * m2, axis=0).reshape(1, 1, 64)

    return pl.pallas_call(
        body,
        grid=(B, N // TN),
        in_specs=[pl.BlockSpec((1, TN, 1), lambda b, j: (0, j, 0)),
                  pl.BlockSpec((70, 64), lambda b, j: (0, 0)),
                  pl.BlockSpec((6, 64), lambda b, j: (0, 0)),
                  pl.BlockSpec((1, TN, 16, 128), lambda b, j: (b, j, 0, 0)),
                  pl.BlockSpec((1, TN, 3), lambda b, j: (b, j, 0)),
                  pl.BlockSpec((1, TN, 64), lambda b, j: (b, j, 0))],
        out_specs=[pl.BlockSpec((1, TN, 16, 64), lambda b, j: (b, j, 0, 0)),
                   pl.BlockSpec((1, TN, 16, 64), lambda b, j: (b, j, 0, 0)),
                   pl.BlockSpec((1, 1, 64), lambda b, j: (0, 0, 0)),
                   pl.BlockSpec((1, 1, 64), lambda b, j: (0, 0, 0))],
        out_shape=[jax.ShapeDtypeStruct((B, N, 16, 64), jnp.float32),
                   jax.ShapeDtypeStruct((B, N, 16, 64), jnp.float32),
                   jax.ShapeDtypeStruct((1, 1, 64), jnp.float32),
                   jax.ShapeDtypeStruct((1, 1, 64), jnp.float32)],
    )(mask3, W1t, Wet, g1, wxlz, wpn)


def _build_pc_pass(g2, wxlz, Wpct, mask3, TN=_TN):
    B, N = g2.shape[0], g2.shape[1]
    R = TN * 16

    def body(mask_ref, w_ref, g_ref, wx_ref, ye_ref, acc_ref):
        m2 = _mask2(mask_ref, TN)
        gxp = g_ref[0][:, :, 64:67]
        wx3 = jnp.broadcast_to(wx_ref[0][:, None, :], (TN, 16, 3))
        diff = gxp - wx3
        euc = jnp.sqrt(jnp.sum(diff * diff, axis=2, keepdims=True) + 1e-20)
        X10 = jnp.concatenate([wx3, gxp, diff, euc], axis=2).reshape(R, 10)
        Y = lax.dot_general(X10, w_ref[...], (((1,), (0,)), ((), ())),
                            preferred_element_type=jnp.float32)
        ye_ref[0] = Y.reshape(TN, 16, 64)

        @pl.when(_first())
        def _():
            acc_ref[...] = jnp.zeros((1, 1, 64), jnp.float32)
        acc_ref[...] += jnp.sum(Y * m2, axis=0).reshape(1, 1, 64)

    return pl.pallas_call(
        body,
        grid=(B, N // TN),
        in_specs=[pl.BlockSpec((1, TN, 1), lambda b, j: (0, j, 0)),
                  pl.BlockSpec((10, 64), lambda b, j: (0, 0)),
                  pl.BlockSpec((1, TN, 16, 128), lambda b, j: (b, j, 0, 0)),
                  pl.BlockSpec((1, TN, 3), lambda b, j: (b, j, 0))],
        out_specs=[pl.BlockSpec((1, TN, 16, 64), lambda b, j: (b, j, 0, 0)),
                   pl.BlockSpec((1, 1, 64), lambda b, j: (0, 0, 0))],
        out_shape=[jax.ShapeDtypeStruct((B, N, 16, 64), jnp.float32),
                   jax.ShapeDtypeStruct((1, 1, 64), jnp.float32)],
    )(mask3, Wpct, g2, wxlz)


def _nc2bnc(feats, batch_info, length):
    counts = jnp.bincount(batch_info, length=length)
    n = feats[0].shape[0]
    n_t = jnp.max(counts)
    offset = jnp.cumsum(counts) - counts
    ind = jnp.arange(n)
    new_count = jnp.full_like(counts, n)
    new_offset = jnp.cumsum(new_count) - new_count
    ind = ind + (new_offset - offset)[batch_info]
    out = []
    for feat in feats:
        c = feat.shape[-1]
        buf = jnp.zeros((length * n, c), dtype=jnp.float32).at[ind].set(feat)
        out.append(buf.reshape(length, n, c))
    return out, ind, n_t


def _norm_ch(x):
    m = jnp.mean(x, axis=-1, keepdims=True)
    s = jnp.std(x, axis=-1, keepdims=True, ddof=1)
    return (x - m) / jnp.maximum(s, 1e-12)


def _mlp_layer(x, p, m, denom):
    W, gamma, beta = p
    y = jnp.einsum('bnkc,oc->bnko', x, W)
    mean = jnp.sum(y * m, axis=(0, 1, 2), keepdims=True) / denom
    var = jnp.sum((y - mean) * (y - mean) * m, axis=(0, 1, 2), keepdims=True) / denom
    y = (y - mean) / jnp.sqrt(var + 1e-5) * gamma + beta
    return jnp.where(y >= 0, y, 0.01 * y)


def kernel(warped_xyz, warped_points, batch_info, batch_size, f2_xyz, f2_points, lidar_z, params):
    B = f2_xyz.shape[0]
    M = f2_xyz.shape[1]
    (wx, wp, lz), inv, n_t = _nc2bnc(
        [warped_xyz, warped_points, lidar_z], batch_info, B)
    valid_mask = (jnp.sum(wx * wx, axis=-1) >= 1e-10).astype(jnp.float32)

    idx_q = _knn_topk(wx, f2_xyz, jnp.zeros((B, M), jnp.float32), n_t, _K)
    bi = jnp.arange(B)[:, None, None]
    C = f2_points.shape[-1]
    n_all = wx.shape[1]
    tab1 = jnp.concatenate(
        [f2_points.reshape(B * M, C),
         jnp.pad(f2_xyz.reshape(B * M, 3), ((0, 0), (0, 125 - C)))], axis=1)
    flat_q = (idx_q + (bi * M)).reshape(-1)
    g1 = _sc_gather(tab1, flat_q).reshape(B, n_all, _K, 128)

    qi_points_grouped = g1[:, :, :, 0:C]
    qi_xyz_grouped = g1[:, :, :, C:C + 3]

    wx = wx * lz
    K = _K
    b, n, _ = wx.shape
    slot_mask = (jnp.arange(n) < n_t).astype(jnp.float32)[None, :, None, None]
    n_f = n_t.astype(jnp.float32)
    denom_q = b * n_f * K
    denom_p = b * n_f * _K
    pi_xyz_expanded = jnp.broadcast_to(wx[:, :, None, :], (b, n, K, 3))
    pi_points_expanded = jnp.broadcast_to(wp[:, :, None, :], (b, n, K, wp.shape[-1]))
    pi_xyz_diff_concat = jnp.concatenate([pi_xyz_expanded, qi_xyz_grouped], axis=3)
    pi_points_expanded = _norm_ch(pi_points_expanded)
    qi_points_grouped = _norm_ch(qi_points_grouped)
    pi_feat_diff = pi_points_expanded * qi_points_grouped
    pi_feat1_new = jnp.concatenate([pi_xyz_diff_concat, pi_feat_diff], axis=3)
    for p in params['mlp1']:
        pi_feat1_new = _mlp_layer(pi_feat1_new, p, slot_mask, denom_q)
    pi_xyz_encoding = _mlp_layer(pi_xyz_diff_concat, params['pi_enc'], slot_mask, denom_q)
    pi_concat = jnp.concatenate([pi_xyz_encoding, pi_feat1_new], axis=3)
    for p in params['mlp2']:
        pi_concat = _mlp_layer(pi_concat, p, slot_mask, denom_q)
    WQ = jax.nn.softmax(pi_concat, axis=2)
    pi_feat1_new = jnp.sum(WQ * pi_feat1_new, axis=2)

    pen = (1.0 - valid_mask) * 1e10
    idx_p = _knn_topk(wx, wx, pen, n_t, _K)
    c2 = pi_feat1_new.shape[-1]
    tab2 = jnp.concatenate(
        [pi_feat1_new.reshape(b * n, c2),
         jnp.pad(wx.reshape(b * n, 3), ((0, 0), (0, 125 - c2)))], axis=1)
    flat_p = (idx_p + (bi * n)).reshape(-1)
    g2 = _sc_gather(tab2, flat_p).reshape(b, n, _K, 128)
    pc_points_grouped = g2[:, :, :, 0:c2]
    pc_xyz_grouped = g2[:, :, :, c2:c2 + 3]

    pc_xyz_new = jnp.broadcast_to(wx[:, :, None, :], (b, n, _K, 3))
    pc_points_new = jnp.broadcast_to(wp[:, :, None, :], (b, n, _K, wp.shape[-1]))
    pc_xyz_diff = pc_xyz_grouped - pc_xyz_new
    pc_euc_diff = jnp.sqrt(jnp.sum(pc_xyz_diff * pc_xyz_diff, axis=3, keepdims=True) + 1e-20)
    pc_xyz_diff_concat = jnp.concatenate(
        [pc_xyz_new, pc_xyz_grouped, pc_xyz_diff, pc_euc_diff], axis=3)
    pc_xyz_encoding = _mlp_layer(pc_xyz_diff_concat, params['pc_enc'], slot_mask, denom_p)
    pc_concat = jnp.concatenate([pc_xyz_encoding, pc_points_new, pc_points_grouped], axis=-1)
    for p in params['mlp2b']:
        pc_concat = _mlp_layer(pc_concat, p, slot_mask, denom_p)
    WP = jax.nn.softmax(pc_concat, axis=2)
    pc_feat1_new = jnp.sum(WP * pc_points_grouped, axis=2)
    c = pc_feat1_new.shape[-1]
    flat_out = jnp.pad(pc_feat1_new.reshape(b * n, c), ((0, 0), (0, 128 - c)))
    return _sc_gather(flat_out, inv.astype(jnp.int32), chunk=256)[:, :c]
